# baseline (speedup 1.0000x reference)
import functools

import jax
import jax.numpy as jnp
from jax import lax
from jax.experimental import pallas as pl
from jax.experimental.pallas import tpu as pltpu
from jax.experimental.pallas import tpu_sc as plsc

DATASET = 100000
D = 128
BATCH = 4096
HIST = 50
HIST_PAD = 56

NC = 2
NS = 16
NW = NC * NS
ROWS_PER_W = BATCH // NW
NBUF = 8
LANES = 16


def _emb_body(idx_hbm, table_hbm, out_hbm, idx_v, rows_v, *sems):
    gsems = sems[:NBUF]
    ssems = sems[NBUF:]
    wid = lax.axis_index("s") * NC + lax.axis_index("c")
    base = wid * ROWS_PER_W

    pltpu.sync_copy(idx_hbm.at[pl.ds(base * HIST_PAD, ROWS_PER_W * HIST_PAD)],
                    idx_v)

    def gather(r, b):
        return pltpu.async_copy(
            table_hbm.at[idx_v.at[pl.ds(r * HIST_PAD, HIST_PAD)]],
            rows_v.at[b], gsems[b])

    def store(r, b):
        return pltpu.async_copy(rows_v.at[b].at[pl.ds(0, HIST)],
                                out_hbm.at[base + r], ssems[b])

    for b in range(NBUF):
        gather(b, b)

    def outer(i, carry):
        for b in range(NBUF):
            r = i * NBUF + b
            buf = rows_v.at[b]
            pltpu.make_async_copy(
                table_hbm.at[idx_v.at[pl.ds(r * HIST_PAD, HIST_PAD)]],
                buf, gsems[b]).wait()

            def relu_row(q, c):
                for j in range(D // LANES):
                    sl = pl.ds(j * LANES, LANES)
                    buf[q, sl] = jnp.maximum(buf[q, sl], 0.0)
                return c

            lax.fori_loop(0, HIST, relu_row, 0)

            store(r, b)

            @pl.when(r + NBUF < ROWS_PER_W)
            def _():
                pltpu.make_async_copy(buf.at[pl.ds(0, HIST)],
                                      out_hbm.at[base + r], ssems[b]).wait()
                gather(r + NBUF, b)

        return carry

    lax.fori_loop(0, ROWS_PER_W // NBUF, outer, 0)

    for b in range(NBUF):
        r = ROWS_PER_W - NBUF + b
        pltpu.make_async_copy(rows_v.at[b].at[pl.ds(0, HIST)],
                              out_hbm.at[base + r], ssems[b]).wait()


def _emb_call(idx_flat, weight):
    mesh = plsc.VectorSubcoreMesh(core_axis_name="c", subcore_axis_name="s")
    fn = functools.partial(
        pl.kernel,
        mesh=mesh,
        out_type=jax.ShapeDtypeStruct((BATCH, HIST, D), jnp.float32),
        scratch_types=[
            pltpu.VMEM((ROWS_PER_W * HIST_PAD,), jnp.int32),
            pltpu.VMEM((NBUF, HIST_PAD, D), jnp.float32),
        ] + [pltpu.SemaphoreType.DMA] * (2 * NBUF),
    )(_emb_body)
    return fn(idx_flat, weight)


def kernel(indices, weight):
    idx_pad = jnp.concatenate(
        [indices, jnp.zeros((BATCH, HIST_PAD - HIST), jnp.int32)], axis=1)
    return _emb_call(idx_pad.reshape(BATCH * HIST_PAD), weight)
